# baseline (speedup 1.0000x reference)
import jax
import jax.numpy as jnp
from jax import lax
from jax.experimental import pallas as pl
from jax.experimental.pallas import tpu as pltpu
from jax.experimental.pallas import tpu_sc as plsc

VOCAB = 1000000
DIM = 64
B = 16384
P = 20
NPAIR = 3 * P
NW = 32
B_PER_W = B // NW
GB = 8
ROWS_G = GB * NPAIR
NG = B_PER_W // GB
OUT_W = B_PER_W * NPAIR
OUT_R = OUT_W // 128
DEPTH = 2


def _sc_body(ctx_idx_hbm, emb_u_hbm, v_emb_hbm, y_hbm,
             ctx_idx_v, rows0_v, rows1_v, ugrp0_v, ugrp1_v,
             tp_v, out_v, sem0, sem1):
    wid = lax.axis_index("s") * 2 + lax.axis_index("c")
    base_b = wid * B_PER_W

    pltpu.sync_copy(ctx_idx_hbm.at[pl.ds(base_b * NPAIR, OUT_W)], ctx_idx_v)

    rows = (rows0_v, rows1_v)
    ugrp = (ugrp0_v, ugrp1_v)
    sems = (sem0, sem1)

    def fill(g, par):
        pltpu.async_copy(emb_u_hbm.at[pl.ds(base_b + g * GB, GB)],
                         ugrp[par], sems[par])
        for q in range(ROWS_G // 120):
            pltpu.async_copy(
                v_emb_hbm.at[ctx_idx_v.at[pl.ds(g * ROWS_G + q * 120, 120)]],
                rows[par].at[pl.ds(q * 120, 120)], sems[par])

    def drain(par):
        pltpu.make_async_copy(emb_u_hbm.at[pl.ds(0, GB)],
                              ugrp[par], sems[par]).wait()
        for q in range(ROWS_G // 120):
            pltpu.make_async_copy(v_emb_hbm.at[pl.ds(0, 120)],
                                  rows[par].at[pl.ds(q * 120, 120)],
                                  sems[par]).wait()

    row_iota = lax.iota(jnp.int32, 16)
    cols = [jnp.full((16,), c, jnp.int32) for c in range(16)]

    fill(0, 0)
    fill(1, 1)

    @pl.loop(0, NG // DEPTH)
    def _trip(t):
        for par in range(DEPTH):
            g = DEPTH * t + par
            drain(par)
            gbase = g * ROWS_G
            u_cache = {}
            for j in range(ROWS_G):
                bb, jj = divmod(j, NPAIR)
                if bb not in u_cache:
                    uc = [ugrp[par][bb, pl.ds(16 * k, 16)] for k in range(4)]
                    u_cache[bb] = (uc, [-c for c in uc])
                uc, nuc = u_cache[bb]
                ch = nuc if jj < P else uc
                r = [rows[par][j, pl.ds(16 * k, 16)] for k in range(4)]
                part = ((r[0] * ch[0] + r[1] * ch[1])
                        + (r[2] * ch[2] + r[3] * ch[3]))
                tp_v[j % 16] = part
                if j % 16 == 15:
                    acc = [plsc.load_gather(tp_v, [row_iota, cols[c]])
                           for c in range(16)]
                    while len(acc) > 1:
                        acc = [acc[i] + acc[i + 1]
                               for i in range(0, len(acc), 2)]
                    out_v[pl.ds(gbase + (j - 15), 16)] = acc[0]

            @pl.when(g + 2 < NG)
            def _prefetch():
                fill(g + 2, par)

    pltpu.sync_copy(out_v, y_hbm.at[pl.ds(wid * OUT_W, OUT_W)])


def _tc_finish(y2d):
    nrows = y2d.shape[0]
    blk = 256
    grid = nrows // blk

    def body(y_ref, o_ref):
        i = pl.program_id(0)
        x = y_ref[...]
        sp = jnp.maximum(x, 0.0) + jnp.log1p(jnp.exp(-jnp.abs(x)))
        s = jnp.sum(sp) * (1.0 / B)

        @pl.when(i == 0)
        def _init():
            o_ref[...] = jnp.zeros_like(o_ref)

        o_ref[...] = o_ref[...] + s

    return pl.pallas_call(
        body,
        grid=(grid,),
        in_specs=[pl.BlockSpec((blk, 128), lambda i: (i, 0))],
        out_specs=pl.BlockSpec((1, 1), lambda i: (0, 0)),
        out_shape=jax.ShapeDtypeStruct((1, 1), jnp.float32),
    )(y2d)


def kernel(u, v, neg, wn, sim, not_sim, mismatch, u_emb, v_emb):
    del sim, not_sim, mismatch
    u_i = u.astype(jnp.int32)
    ctx = jnp.concatenate([v, neg, wn], axis=1).astype(jnp.int32).reshape(-1)
    emb_u = jnp.take(u_emb, u_i, axis=0)

    mesh = plsc.VectorSubcoreMesh(core_axis_name="c", subcore_axis_name="s")
    y2d = pl.kernel(
        _sc_body,
        out_type=jax.ShapeDtypeStruct((B * NPAIR,), jnp.float32),
        mesh=mesh,
        compiler_params=pltpu.CompilerParams(
            needs_layout_passes=False, use_tc_tiling_on_sc=False),
        scratch_types=[
            pltpu.VMEM((OUT_W,), jnp.int32),
            pltpu.VMEM((ROWS_G, DIM), jnp.float32),
            pltpu.VMEM((ROWS_G, DIM), jnp.float32),
            pltpu.VMEM((GB, DIM), jnp.float32),
            pltpu.VMEM((GB, DIM), jnp.float32),
            pltpu.VMEM((16, 16), jnp.float32),
            pltpu.VMEM((OUT_W,), jnp.float32),
            pltpu.SemaphoreType.DMA,
            pltpu.SemaphoreType.DMA,
        ],
    )(ctx, emb_u, v_emb)

    loss = _tc_finish(y2d.reshape(B * NPAIR // 128, 128))
    return loss[0, 0]


# speedup vs baseline: 1.0774x; 1.0774x over previous
import jax
import jax.numpy as jnp
from jax import lax
from jax.experimental import pallas as pl
from jax.experimental.pallas import tpu as pltpu
from jax.experimental.pallas import tpu_sc as plsc

VOCAB = 1000000
DIM = 64
B = 16384
P = 20
NPAIR = 3 * P
NW = 32
B_PER_W = B // NW
GB = 2
ROWS_G = GB * NPAIR
NG = B_PER_W // GB
OUT_W = B_PER_W * NPAIR
OUT_R = OUT_W // 128
DEPTH = 2


def _sc_body(ctx_idx_hbm, emb_u_hbm, v_emb_hbm, y_hbm,
             ctx_idx_v, rows0_v, rows1_v, ugrp0_v, ugrp1_v,
             tp_v, out_v, sem0, sem1):
    wid = lax.axis_index("s") * 2 + lax.axis_index("c")
    base_b = wid * B_PER_W

    pltpu.sync_copy(ctx_idx_hbm.at[pl.ds(base_b * NPAIR, OUT_W)], ctx_idx_v)

    rows = (rows0_v, rows1_v)
    ugrp = (ugrp0_v, ugrp1_v)
    sems = (sem0, sem1)

    def fill(g, par):
        pltpu.async_copy(emb_u_hbm.at[pl.ds(base_b + g * GB, GB)],
                         ugrp[par], sems[par])
        for q in range(ROWS_G // 120):
            pltpu.async_copy(
                v_emb_hbm.at[ctx_idx_v.at[pl.ds(g * ROWS_G + q * 120, 120)]],
                rows[par].at[pl.ds(q * 120, 120)], sems[par])

    def drain(par):
        pltpu.make_async_copy(emb_u_hbm.at[pl.ds(0, GB)],
                              ugrp[par], sems[par]).wait()
        for q in range(ROWS_G // 120):
            pltpu.make_async_copy(v_emb_hbm.at[pl.ds(0, 120)],
                                  rows[par].at[pl.ds(q * 120, 120)],
                                  sems[par]).wait()

    row_iota = lax.iota(jnp.int32, 16)
    cols = [jnp.full((16,), c, jnp.int32) for c in range(16)]

    fill(0, 0)
    fill(1, 1)

    @pl.loop(0, NG // DEPTH)
    def _trip(t):
        for par in range(DEPTH):
            g = DEPTH * t + par
            drain(par)
            gbase = g * ROWS_G
            u_cache = {}
            for j in range(ROWS_G):
                bb, jj = divmod(j, NPAIR)
                if bb not in u_cache:
                    uc = [ugrp[par][bb, pl.ds(16 * k, 16)] for k in range(4)]
                    u_cache[bb] = (uc, [-c for c in uc])
                uc, nuc = u_cache[bb]
                ch = nuc if jj < P else uc
                r = [rows[par][j, pl.ds(16 * k, 16)] for k in range(4)]
                part = ((r[0] * ch[0] + r[1] * ch[1])
                        + (r[2] * ch[2] + r[3] * ch[3]))
                tp_v[j % 16] = part
                if j % 16 == 15:
                    acc = [plsc.load_gather(tp_v, [row_iota, cols[c]])
                           for c in range(16)]
                    while len(acc) > 1:
                        acc = [acc[i] + acc[i + 1]
                               for i in range(0, len(acc), 2)]
                    out_v[pl.ds(gbase + (j - 15), 16)] = acc[0]

            @pl.when(g + 2 < NG)
            def _prefetch():
                fill(g + 2, par)

    pltpu.sync_copy(out_v, y_hbm.at[pl.ds(wid * OUT_W, OUT_W)])


def _tc_finish(y2d):
    nrows = y2d.shape[0]
    blk = 256
    grid = nrows // blk

    def body(y_ref, o_ref):
        i = pl.program_id(0)
        x = y_ref[...]
        sp = jnp.maximum(x, 0.0) + jnp.log1p(jnp.exp(-jnp.abs(x)))
        s = jnp.sum(sp) * (1.0 / B)

        @pl.when(i == 0)
        def _init():
            o_ref[...] = jnp.zeros_like(o_ref)

        o_ref[...] = o_ref[...] + s

    return pl.pallas_call(
        body,
        grid=(grid,),
        in_specs=[pl.BlockSpec((blk, 128), lambda i: (i, 0))],
        out_specs=pl.BlockSpec((1, 1), lambda i: (0, 0)),
        out_shape=jax.ShapeDtypeStruct((1, 1), jnp.float32),
    )(y2d)


def kernel(u, v, neg, wn, sim, not_sim, mismatch, u_emb, v_emb):
    del sim, not_sim, mismatch
    u_i = u.astype(jnp.int32)
    ctx = jnp.concatenate([v, neg, wn], axis=1).astype(jnp.int32).reshape(-1)
    emb_u = jnp.take(u_emb, u_i, axis=0)

    mesh = plsc.VectorSubcoreMesh(core_axis_name="c", subcore_axis_name="s")
    y2d = pl.kernel(
        _sc_body,
        out_type=jax.ShapeDtypeStruct((B * NPAIR,), jnp.float32),
        mesh=mesh,
        compiler_params=pltpu.CompilerParams(
            needs_layout_passes=False, use_tc_tiling_on_sc=False),
        scratch_types=[
            pltpu.VMEM((OUT_W,), jnp.int32),
            pltpu.VMEM((ROWS_G, DIM), jnp.float32),
            pltpu.VMEM((ROWS_G, DIM), jnp.float32),
            pltpu.VMEM((GB, DIM), jnp.float32),
            pltpu.VMEM((GB, DIM), jnp.float32),
            pltpu.VMEM((16, 16), jnp.float32),
            pltpu.VMEM((OUT_W,), jnp.float32),
            pltpu.SemaphoreType.DMA,
            pltpu.SemaphoreType.DMA,
        ],
    )(ctx, emb_u, v_emb)

    loss = _tc_finish(y2d.reshape(B * NPAIR // 128, 128))
    return loss[0, 0]
